# Initial kernel scaffold; baseline (speedup 1.0000x reference)
import jax
import jax.numpy as jnp
from jax.experimental import pallas as pl


def kernel(x, W_theta, W_o, concept_pool, gamma):
    raise NotImplementedError("write your pallas kernel here")


# speedup vs baseline: 2.5165x; 2.5165x over previous
import functools

import jax
import jax.numpy as jnp
from jax.experimental import pallas as pl


def _attn_block(x_ref, wt_ref, wo_ref, pool_ref, gamma_ref, out_ref):
    xb = x_ref[0]
    pool = pool_ref[:]
    theta = jax.lax.dot_general(
        wt_ref[:], xb, (((1,), (0,)), ((), ())),
        preferred_element_type=jnp.float32)
    logits = jax.lax.dot_general(
        pool, theta, (((0,), (0,)), ((), ())),
        preferred_element_type=jnp.float32)
    m = jnp.max(logits, axis=0, keepdims=True)
    e = jnp.exp(logits - m)
    s = jnp.sum(e, axis=0, keepdims=True)
    attn = e / s
    agg = jax.lax.dot_general(
        pool, attn, (((1,), (0,)), ((), ())),
        preferred_element_type=jnp.float32)
    o = jax.lax.dot_general(
        wo_ref[:], agg, (((1,), (0,)), ((), ())),
        preferred_element_type=jnp.float32)
    out_ref[0] = gamma_ref[0, 0] * o + xb


@functools.partial(jax.jit, static_argnames=("n_blk",))
def _run(x, W_theta, W_o, concept_pool, gamma, n_blk=256):
    B, C, H, W = x.shape
    fd, P = concept_pool.shape
    n = H * W
    xr = x.reshape(B, C, n)
    grid = (B, n // n_blk)
    out = pl.pallas_call(
        _attn_block,
        grid=grid,
        in_specs=[
            pl.BlockSpec((1, C, n_blk), lambda b, j: (b, 0, j)),
            pl.BlockSpec((fd, C), lambda b, j: (0, 0)),
            pl.BlockSpec((C, fd), lambda b, j: (0, 0)),
            pl.BlockSpec((fd, P), lambda b, j: (0, 0)),
            pl.BlockSpec((1, 1), lambda b, j: (0, 0)),
        ],
        out_specs=pl.BlockSpec((1, C, n_blk), lambda b, j: (b, 0, j)),
        out_shape=jax.ShapeDtypeStruct((B, C, n), jnp.float32),
    )(xr, W_theta, W_o, concept_pool, jnp.reshape(gamma, (1, 1)))
    return out.reshape(B, C, H, W)


def kernel(x, W_theta, W_o, concept_pool, gamma):
    return _run(x, W_theta, W_o, concept_pool, gamma)


# speedup vs baseline: 3.0342x; 1.2058x over previous
import functools

import jax
import jax.numpy as jnp
from jax.experimental import pallas as pl


def _attn_block(x_ref, wt_ref, wo_ref, pool_ref, gamma_ref, out_ref):
    xb = x_ref[0]
    pool = pool_ref[:]
    theta = jax.lax.dot_general(
        wt_ref[:], xb, (((1,), (0,)), ((), ())),
        preferred_element_type=jnp.float32)
    logits = jax.lax.dot_general(
        pool, theta, (((0,), (0,)), ((), ())),
        preferred_element_type=jnp.float32)
    m = jnp.max(logits, axis=0, keepdims=True)
    e = jnp.exp(logits - m)
    s = jnp.sum(e, axis=0, keepdims=True)
    agg = jax.lax.dot_general(
        pool, e, (((1,), (0,)), ((), ())),
        preferred_element_type=jnp.float32) / s
    o = jax.lax.dot_general(
        wo_ref[:], agg, (((1,), (0,)), ((), ())),
        preferred_element_type=jnp.float32)
    out_ref[0] = gamma_ref[0, 0] * o + xb


@functools.partial(jax.jit, static_argnames=("n_blk",))
def _run(x, W_theta, W_o, concept_pool, gamma, n_blk=256):
    B, C, H, W = x.shape
    fd, P = concept_pool.shape
    n = H * W
    xr = x.reshape(B, C, n)
    grid = (B, n // n_blk)
    out = pl.pallas_call(
        _attn_block,
        grid=grid,
        in_specs=[
            pl.BlockSpec((1, C, n_blk), lambda b, j: (b, 0, j)),
            pl.BlockSpec((fd, C), lambda b, j: (0, 0)),
            pl.BlockSpec((C, fd), lambda b, j: (0, 0)),
            pl.BlockSpec((fd, P), lambda b, j: (0, 0)),
            pl.BlockSpec((1, 1), lambda b, j: (0, 0)),
        ],
        out_specs=pl.BlockSpec((1, C, n_blk), lambda b, j: (b, 0, j)),
        out_shape=jax.ShapeDtypeStruct((B, C, n), jnp.float32),
    )(xr, W_theta, W_o, concept_pool, jnp.reshape(gamma, (1, 1)))
    return out.reshape(B, C, H, W)


def kernel(x, W_theta, W_o, concept_pool, gamma):
    return _run(x, W_theta, W_o, concept_pool, gamma)
